# Initial kernel scaffold; baseline (speedup 1.0000x reference)
import jax
import jax.numpy as jnp
from jax.experimental import pallas as pl


def kernel(inpTok, tok_table, pos_table):
    raise NotImplementedError("write your pallas kernel here")


# speedup vs baseline: 2.5149x; 2.5149x over previous
import functools

import jax
import jax.numpy as jnp
from jax import lax
from jax.experimental import pallas as pl
from jax.experimental.pallas import tpu as pltpu
from jax.experimental.pallas import tpu_sc as plsc

VOC = 100000
D = 128
T = 100
B = 16384
NC = 2
NS = 16
NW = NC * NS
SEQ_PER_W = B // NW
S_CH = 4
N_CHUNKS = SEQ_PER_W // S_CH


def _body(tok_hbm, pos_hbm, idx_hbm, out_hbm, idx_v, rows_v, sem):
    wid = lax.axis_index("s") * NC + lax.axis_index("c")
    wbase = wid * SEQ_PER_W

    def chunk(c, carry):
        seq0 = wbase + c * S_CH
        pltpu.sync_copy(idx_hbm.at[pl.ds(seq0, S_CH)], idx_v)
        fills = [pltpu.async_copy(pos_hbm, rows_v.at[j], sem)
                 for j in range(S_CH)]
        for f in fills:
            f.wait()
        gathers = [pltpu.async_copy(tok_hbm.at[idx_v.at[j]], rows_v.at[j],
                                    sem, add=True)
                   for j in range(S_CH)]
        for g in gathers:
            g.wait()
        pltpu.sync_copy(rows_v, out_hbm.at[pl.ds(seq0, S_CH)])
        return carry

    lax.fori_loop(0, N_CHUNKS, chunk, 0)


@functools.partial(jax.jit, static_argnums=())
def _emb(tok_table, pos_table, idx):
    grid_kernel = pl.kernel(
        _body,
        out_type=jax.ShapeDtypeStruct((B, T, D), jnp.float32),
        mesh=plsc.VectorSubcoreMesh(
            core_axis_name="c", subcore_axis_name="s",
            num_cores=NC, num_subcores=NS),
        scratch_types=[
            pltpu.VMEM((S_CH, T), jnp.int32),
            pltpu.VMEM((S_CH, T, D), jnp.float32),
            pltpu.SemaphoreType.DMA,
        ],
    )
    return grid_kernel(tok_table, pos_table, idx)


def kernel(inpTok, tok_table, pos_table):
    return _emb(tok_table, pos_table, inpTok.astype(jnp.int32))
